# Initial kernel scaffold; baseline (speedup 1.0000x reference)
import jax
import jax.numpy as jnp
from jax.experimental import pallas as pl


def kernel(x, edge_index, W1, b1, g1, be1, W2, b2, g2, be2, W3, b3):
    raise NotImplementedError("write your pallas kernel here")


# speedup vs baseline: 13.7118x; 13.7118x over previous
import functools

import jax
import jax.numpy as jnp
from jax import lax
from jax.experimental import pallas as pl
from jax.experimental.pallas import tpu as pltpu
from jax.experimental.pallas import tpu_sc as plsc

N = 10000
D = 128
E = 320000
EPS = 1e-5

NC = 2
NS = 16
NW = NC * NS
CHUNK = 128
NCHUNK = E // CHUNK
NPAD = 10240
ROWS_PER_TILE = NPAD // NS
WB = 128

BLK = 1000
GRID = N // BLK


def _sc_agg_body(h_hbm, src_hbm, dst_hbm, out_hbm,
                 acc, src_v, dst_v, rows_v, wb_v, sem):
    cid = lax.axis_index("c")
    sid = lax.axis_index("s")
    wid = cid * NS + sid

    def zrow(j, carry):
        for k in range(D // 16):
            wb_v[j, pl.ds(k * 16, 16)] = jnp.zeros((16,), jnp.float32)
        return carry
    lax.fori_loop(0, WB, zrow, 0)
    base = sid * ROWS_PER_TILE
    for j in range(ROWS_PER_TILE // WB):
        pltpu.sync_copy(wb_v, acc.at[pl.ds(base + j * WB, WB)])
    plsc.subcore_barrier()

    nchunks = (NCHUNK - wid + NW - 1) // NW

    def body(i, carry):
        ebase = (wid + i * NW) * CHUNK
        pltpu.sync_copy(src_hbm.at[pl.ds(ebase, CHUNK)], src_v)
        pltpu.sync_copy(dst_hbm.at[pl.ds(ebase, CHUNK)], dst_v)
        pltpu.async_copy(h_hbm.at[src_v], rows_v, sem).wait()
        pltpu.sync_copy(rows_v, acc.at[dst_v], add=True)
        return carry
    lax.fori_loop(0, nchunks, body, 0)
    plsc.subcore_barrier()

    for j in range(ROWS_PER_TILE // WB):
        r = base + j * WB
        pltpu.sync_copy(acc.at[pl.ds(r, WB)], wb_v)
        pltpu.sync_copy(wb_v, out_hbm.at[cid, pl.ds(r, WB)])


def _sc_deg_body(dst_hbm, out_hbm, accd, dst_v, ones_v, wbd_v):
    cid = lax.axis_index("c")
    sid = lax.axis_index("s")
    wid = cid * NS + sid

    def fill_ones(j, carry):
        ones_v[j, pl.ds(0, 16)] = jnp.ones((16,), jnp.float32)
        return carry
    lax.fori_loop(0, CHUNK, fill_ones, 0)

    def zrow(j, carry):
        wbd_v[j, pl.ds(0, 16)] = jnp.zeros((16,), jnp.float32)
        return carry
    lax.fori_loop(0, WB, zrow, 0)
    base = sid * ROWS_PER_TILE
    for j in range(ROWS_PER_TILE // WB):
        pltpu.sync_copy(wbd_v, accd.at[pl.ds(base + j * WB, WB)])
    plsc.subcore_barrier()

    nchunks = (NCHUNK - wid + NW - 1) // NW

    def body(i, carry):
        ebase = (wid + i * NW) * CHUNK
        pltpu.sync_copy(dst_hbm.at[pl.ds(ebase, CHUNK)], dst_v)
        pltpu.sync_copy(ones_v, accd.at[dst_v], add=True)
        return carry
    lax.fori_loop(0, nchunks, body, 0)
    plsc.subcore_barrier()

    for j in range(ROWS_PER_TILE // WB):
        r = base + j * WB
        pltpu.sync_copy(accd.at[pl.ds(r, WB)], wbd_v)
        pltpu.sync_copy(wbd_v, out_hbm.at[cid, pl.ds(r, WB)])


@functools.lru_cache(maxsize=None)
def _sc_kernels():
    mesh = plsc.VectorSubcoreMesh(core_axis_name="c", subcore_axis_name="s")
    agg = pl.kernel(
        _sc_agg_body,
        out_type=jax.ShapeDtypeStruct((NC, NPAD, D), jnp.float32),
        mesh=mesh,
        scratch_types=[
            pltpu.VMEM_SHARED((NPAD, D), jnp.float32),
            pltpu.VMEM((CHUNK,), jnp.int32),
            pltpu.VMEM((CHUNK,), jnp.int32),
            pltpu.VMEM((CHUNK, D), jnp.float32),
            pltpu.VMEM((WB, D), jnp.float32),
            pltpu.SemaphoreType.DMA,
        ],
    )
    deg = pl.kernel(
        _sc_deg_body,
        out_type=jax.ShapeDtypeStruct((NC, NPAD, 16), jnp.float32),
        mesh=mesh,
        scratch_types=[
            pltpu.VMEM_SHARED((NPAD, 16), jnp.float32),
            pltpu.VMEM((CHUNK,), jnp.int32),
            pltpu.VMEM((CHUNK, 16), jnp.float32),
            pltpu.VMEM((WB, 16), jnp.float32),
        ],
    )
    return agg, deg


def _dinv(deg_ref):
    return lax.rsqrt(1.0 + deg_ref[:, :1])


def _first_mm_body(x_ref, w_ref, d0_ref, d1_ref, o_ref, deg_ref):
    deg_ref[...] = d0_ref[...] + d1_ref[...]
    h = jnp.dot(x_ref[...], w_ref[...], preferred_element_type=jnp.float32)
    o_ref[...] = h * _dinv(deg_ref)


def _comb_body(p0_ref, p1_ref, h_ref, deg_ref, b_ref, z_ref, st_ref):
    i = pl.program_id(0)
    z = (p0_ref[...] + p1_ref[...] + h_ref[...]) * _dinv(deg_ref) + b_ref[...]
    z_ref[...] = z

    @pl.when(i == 0)
    def _():
        st_ref[...] = jnp.zeros_like(st_ref)
    s1 = jnp.sum(z, axis=0, keepdims=True)
    s2 = jnp.sum(z * z, axis=0, keepdims=True)
    st_ref[...] += jnp.concatenate(
        [s1, s2, jnp.zeros((6, D), jnp.float32)], axis=0)


def _bn_mm_body(z_ref, st_ref, g_ref, be_ref, w_ref, deg_ref, o_ref):
    mu = st_ref[0:1, :] * (1.0 / N)
    var = st_ref[1:2, :] * (1.0 / N) - mu * mu
    rstd = lax.rsqrt(var + EPS)
    zn = (z_ref[...] - mu) * rstd * g_ref[...] + be_ref[...]
    zn = jnp.maximum(zn, 0.0)
    h = jnp.dot(zn, w_ref[...], preferred_element_type=jnp.float32)
    o_ref[...] = h * _dinv(deg_ref)


def _final_body(p0_ref, p1_ref, h_ref, deg_ref, b_ref, o_ref):
    o_ref[...] = ((p0_ref[...] + p1_ref[...] + h_ref[...]) * _dinv(deg_ref)
                  + b_ref[...])


def _row_spec():
    return pl.BlockSpec((BLK, D), lambda i: (i, 0))


def _deg_spec():
    return pl.BlockSpec((BLK, 16), lambda i: (i, 0))


def _full_spec(rows):
    return pl.BlockSpec((rows, D), lambda i: (0, 0))


def _first_mm(x, w, d0, d1):
    return pl.pallas_call(
        _first_mm_body,
        grid=(GRID,),
        in_specs=[_row_spec(), _full_spec(D), _deg_spec(), _deg_spec()],
        out_specs=[_row_spec(), _deg_spec()],
        out_shape=[jax.ShapeDtypeStruct((N, D), jnp.float32),
                   jax.ShapeDtypeStruct((N, 16), jnp.float32)],
    )(x, w, d0, d1)


def _comb(p0, p1, h, deg, b):
    return pl.pallas_call(
        _comb_body,
        grid=(GRID,),
        in_specs=[_row_spec(), _row_spec(), _row_spec(), _deg_spec(),
                  _full_spec(1)],
        out_specs=[_row_spec(), _full_spec(8)],
        out_shape=[jax.ShapeDtypeStruct((N, D), jnp.float32),
                   jax.ShapeDtypeStruct((8, D), jnp.float32)],
    )(p0, p1, h, deg, b)


def _bn_mm(z, st, g, be, w, deg):
    return pl.pallas_call(
        _bn_mm_body,
        grid=(GRID,),
        in_specs=[_row_spec(), _full_spec(8), _full_spec(1), _full_spec(1),
                  _full_spec(D), _deg_spec()],
        out_specs=_row_spec(),
        out_shape=jax.ShapeDtypeStruct((N, D), jnp.float32),
    )(z, st, g, be, w, deg)


def _final(p0, p1, h, deg, b):
    return pl.pallas_call(
        _final_body,
        grid=(GRID,),
        in_specs=[_row_spec(), _row_spec(), _row_spec(), _deg_spec(),
                  _full_spec(1)],
        out_specs=_row_spec(),
        out_shape=jax.ShapeDtypeStruct((N, D), jnp.float32),
    )(p0, p1, h, deg, b)


def kernel(x, edge_index, W1, b1, g1, be1, W2, b2, g2, be2, W3, b3):
    agg, degk = _sc_kernels()
    src = edge_index[0]
    dst = edge_index[1]

    deg2 = degk(dst)

    h1, deg = _first_mm(x, W1, deg2[0, :N], deg2[1, :N])
    p = agg(h1, src, dst)
    z1, st1 = _comb(p[0, :N], p[1, :N], h1, deg, b1.reshape(1, D))

    h2 = _bn_mm(z1, st1, g1.reshape(1, D), be1.reshape(1, D), W2, deg)
    p = agg(h2, src, dst)
    z2, st2 = _comb(p[0, :N], p[1, :N], h2, deg, b2.reshape(1, D))

    h3 = _bn_mm(z2, st2, g2.reshape(1, D), be2.reshape(1, D), W3, deg)
    p = agg(h3, src, dst)
    return _final(p[0, :N], p[1, :N], h3, deg, b3.reshape(1, D))


# speedup vs baseline: 24.5573x; 1.7910x over previous
import functools

import jax
import jax.numpy as jnp
from jax import lax
from jax.experimental import pallas as pl
from jax.experimental.pallas import tpu as pltpu
from jax.experimental.pallas import tpu_sc as plsc

N = 10000
D = 128
E = 320000
EPS = 1e-5

NC = 2
NS = 16
NW = NC * NS
ECH = 125
CPT = (E // NW) // ECH
GC = 8
NGRP = CPT // GC
IR = 4
NPAD = 10240
ROWS_PER_TILE = NPAD // NS
WB = 64
ROWB = ECH * D * 4

DCH = 128
NDCH = E // DCH

BLK = 1000
GRID = N // BLK


def _sc_agg_body(h_hbm, src_hbm, dst_hbm, dum_hbm, out_hbm,
                 acc, sidx_v, didx_v, sbuf0, sbuf1, dbuf0, dbuf1,
                 rows0, rows1, wb_v, isem, gs0, gs1, ss0, ss1):
    gsem = (gs0, gs1)
    ssem = (ss0, ss1)
    rows = (rows0, rows1)
    sbuf = (sbuf0, sbuf1)
    dbuf = (dbuf0, dbuf1)
    cid = lax.axis_index("c")
    sid = lax.axis_index("s")
    wid = cid * NS + sid
    iblk = wid * CPT

    pltpu.async_copy(src_hbm.at[pl.ds(iblk, GC)], sidx_v.at[pl.ds(0, GC)],
                     isem)
    pltpu.async_copy(dst_hbm.at[pl.ds(iblk, GC)], didx_v.at[pl.ds(0, GC)],
                     isem)
    pltpu.async_copy(dst_hbm.at[pl.ds(E // ECH, GC)],
                     didx_v.at[pl.ds(2 * GC, GC)], isem)

    def zrow(j, carry):
        for k in range(D // 16):
            wb_v[j, pl.ds(k * 16, 16)] = jnp.zeros((16,), jnp.float32)
        return carry
    lax.fori_loop(0, WB, zrow, 0)
    base = sid * ROWS_PER_TILE
    for j in range(ROWS_PER_TILE // WB):
        pltpu.sync_copy(wb_v, acc.at[pl.ds(base + j * WB, WB)])

    def iwait(ref):
        pltpu.make_async_copy(src_hbm.at[pl.ds(0, GC)], ref.at[pl.ds(0, GC)],
                              isem).wait()
    iwait(sidx_v)
    iwait(didx_v)
    iwait(didx_v)

    def stage(buf, ref, row):
        for kk in range(GC):
            o = min(kk * 16, ECH - 16)
            buf[pl.ds(o, 16)] = ref[row, pl.ds(o, 16)]

    stage(dbuf0, didx_v, 2 * GC)
    stage(dbuf1, didx_v, 2 * GC)
    stage(sbuf0, sidx_v, 0)
    stage(sbuf1, sidx_v, 0)
    plsc.subcore_barrier()

    def gather(rb):
        pltpu.async_copy(h_hbm.at[sbuf[rb]], rows[rb], gsem[rb])

    def scatter(rb):
        pltpu.async_copy(rows[rb], acc.at[dbuf[rb]], ssem[rb], add=True)

    def gwait(rb):
        pltpu.make_async_copy(dum_hbm, rows[rb], gsem[rb]).wait()

    def swait(rb):
        pltpu.make_async_copy(dum_hbm, rows[rb], ssem[rb]).wait()

    gather(1)
    scatter(0)

    def grp(g, carry):
        sb = (g & 1) * GC
        pb = GC - sb
        gn = jnp.minimum(g + 1, NGRP - 1)
        for k in range(GC):
            b = k & 1
            swait(b)
            if k == 1:
                pltpu.async_copy(src_hbm.at[pl.ds(iblk + GC * gn, GC)],
                                 sidx_v.at[pl.ds(pb, GC)], isem)
                pltpu.async_copy(dst_hbm.at[pl.ds(iblk + GC * gn, GC)],
                                 didx_v.at[pl.ds(pb, GC)], isem)
            stage(sbuf[b], sidx_v, sb + k)
            stage(dbuf[b], didx_v, sb + k)
            gather(b)
            gwait(1 - b)
            scatter(1 - b)
            if k == GC - 1:
                iwait(sidx_v)
                iwait(didx_v)
        return carry
    lax.fori_loop(0, NGRP, grp, 0)

    gwait(1)
    scatter(1)
    swait(0)
    swait(1)
    plsc.subcore_barrier()

    for j in range(ROWS_PER_TILE // WB):
        r = base + j * WB
        pltpu.sync_copy(acc.at[pl.ds(r, WB)], wb_v)
        pltpu.sync_copy(wb_v, out_hbm.at[cid, pl.ds(r, WB)])


def _sc_deg_body(dst_hbm, out_hbm, accd, dst_v, ones_v, wbd_v):
    cid = lax.axis_index("c")
    sid = lax.axis_index("s")
    wid = cid * NS + sid

    def fill_ones(j, carry):
        ones_v[j, pl.ds(0, 16)] = jnp.ones((16,), jnp.float32)
        return carry
    lax.fori_loop(0, DCH, fill_ones, 0)

    def zrow(j, carry):
        wbd_v[j, pl.ds(0, 16)] = jnp.zeros((16,), jnp.float32)
        return carry
    lax.fori_loop(0, WB, zrow, 0)
    base = sid * ROWS_PER_TILE
    for j in range(ROWS_PER_TILE // WB):
        pltpu.sync_copy(wbd_v, accd.at[pl.ds(base + j * WB, WB)])
    plsc.subcore_barrier()

    nchunks = (NDCH - wid + NW - 1) // NW

    def body(i, carry):
        ebase = (wid + i * NW) * DCH
        pltpu.sync_copy(dst_hbm.at[pl.ds(ebase, DCH)], dst_v)
        pltpu.sync_copy(ones_v, accd.at[dst_v], add=True)
        return carry
    lax.fori_loop(0, nchunks, body, 0)
    plsc.subcore_barrier()

    for j in range(ROWS_PER_TILE // WB):
        r = base + j * WB
        pltpu.sync_copy(accd.at[pl.ds(r, WB)], wbd_v)
        pltpu.sync_copy(wbd_v, out_hbm.at[cid, pl.ds(r, WB)])


@functools.lru_cache(maxsize=None)
def _sc_kernels():
    mesh = plsc.VectorSubcoreMesh(core_axis_name="c", subcore_axis_name="s")
    agg = pl.kernel(
        _sc_agg_body,
        out_type=jax.ShapeDtypeStruct((NC, NPAD, D), jnp.float32),
        mesh=mesh,
        scratch_types=[
            pltpu.VMEM_SHARED((NPAD, D), jnp.float32),
            pltpu.VMEM((2 * GC, ECH), jnp.int32),
            pltpu.VMEM((3 * GC, ECH), jnp.int32),
            pltpu.VMEM((ECH,), jnp.int32),
            pltpu.VMEM((ECH,), jnp.int32),
            pltpu.VMEM((ECH,), jnp.int32),
            pltpu.VMEM((ECH,), jnp.int32),
            pltpu.VMEM((ECH, D), jnp.float32),
            pltpu.VMEM((ECH, D), jnp.float32),
            pltpu.VMEM((WB, D), jnp.float32),
        ] + [pltpu.SemaphoreType.DMA] * 5,
    )
    deg = pl.kernel(
        _sc_deg_body,
        out_type=jax.ShapeDtypeStruct((NC, NPAD, 16), jnp.float32),
        mesh=mesh,
        scratch_types=[
            pltpu.VMEM_SHARED((NPAD, 16), jnp.float32),
            pltpu.VMEM((DCH,), jnp.int32),
            pltpu.VMEM((DCH, 16), jnp.float32),
            pltpu.VMEM((WB, 16), jnp.float32),
        ],
    )
    return agg, deg


def _dinv(deg_ref):
    return lax.rsqrt(1.0 + deg_ref[:, :1])


def _first_mm_body(x_ref, w_ref, d0_ref, d1_ref, o_ref, deg_ref):
    deg_ref[...] = d0_ref[...] + d1_ref[...]
    h = jnp.dot(x_ref[...], w_ref[...], preferred_element_type=jnp.float32)
    o_ref[...] = h * _dinv(deg_ref)


def _comb_body(p0_ref, p1_ref, h_ref, deg_ref, b_ref, z_ref, st_ref):
    i = pl.program_id(0)
    z = (p0_ref[...] + p1_ref[...] + h_ref[...]) * _dinv(deg_ref) + b_ref[...]
    z_ref[...] = z

    @pl.when(i == 0)
    def _():
        st_ref[...] = jnp.zeros_like(st_ref)
    s1 = jnp.sum(z, axis=0, keepdims=True)
    s2 = jnp.sum(z * z, axis=0, keepdims=True)
    st_ref[...] += jnp.concatenate(
        [s1, s2, jnp.zeros((6, D), jnp.float32)], axis=0)


def _bn_mm_body(z_ref, st_ref, g_ref, be_ref, w_ref, deg_ref, o_ref):
    mu = st_ref[0:1, :] * (1.0 / N)
    var = st_ref[1:2, :] * (1.0 / N) - mu * mu
    rstd = lax.rsqrt(var + EPS)
    zn = (z_ref[...] - mu) * rstd * g_ref[...] + be_ref[...]
    zn = jnp.maximum(zn, 0.0)
    h = jnp.dot(zn, w_ref[...], preferred_element_type=jnp.float32)
    o_ref[...] = h * _dinv(deg_ref)


def _final_body(p0_ref, p1_ref, h_ref, deg_ref, b_ref, o_ref):
    o_ref[...] = ((p0_ref[...] + p1_ref[...] + h_ref[...]) * _dinv(deg_ref)
                  + b_ref[...])


def _row_spec():
    return pl.BlockSpec((BLK, D), lambda i: (i, 0))


def _deg_spec():
    return pl.BlockSpec((BLK, 16), lambda i: (i, 0))


def _full_spec(rows):
    return pl.BlockSpec((rows, D), lambda i: (0, 0))


def _first_mm(x, w, d0, d1):
    return pl.pallas_call(
        _first_mm_body,
        grid=(GRID,),
        in_specs=[_row_spec(), _full_spec(D), _deg_spec(), _deg_spec()],
        out_specs=[_row_spec(), _deg_spec()],
        out_shape=[jax.ShapeDtypeStruct((N, D), jnp.float32),
                   jax.ShapeDtypeStruct((N, 16), jnp.float32)],
    )(x, w, d0, d1)


def _comb(p0, p1, h, deg, b):
    return pl.pallas_call(
        _comb_body,
        grid=(GRID,),
        in_specs=[_row_spec(), _row_spec(), _row_spec(), _deg_spec(),
                  _full_spec(1)],
        out_specs=[_row_spec(), _full_spec(8)],
        out_shape=[jax.ShapeDtypeStruct((N, D), jnp.float32),
                   jax.ShapeDtypeStruct((8, D), jnp.float32)],
    )(p0, p1, h, deg, b)


def _bn_mm(z, st, g, be, w, deg):
    return pl.pallas_call(
        _bn_mm_body,
        grid=(GRID,),
        in_specs=[_row_spec(), _full_spec(8), _full_spec(1), _full_spec(1),
                  _full_spec(D), _deg_spec()],
        out_specs=_row_spec(),
        out_shape=jax.ShapeDtypeStruct((N, D), jnp.float32),
    )(z, st, g, be, w, deg)


def _final(p0, p1, h, deg, b):
    return pl.pallas_call(
        _final_body,
        grid=(GRID,),
        in_specs=[_row_spec(), _row_spec(), _row_spec(), _deg_spec(),
                  _full_spec(1)],
        out_specs=_row_spec(),
        out_shape=jax.ShapeDtypeStruct((N, D), jnp.float32),
    )(p0, p1, h, deg, b)


def kernel(x, edge_index, W1, b1, g1, be1, W2, b2, g2, be2, W3, b3):
    agg, degk = _sc_kernels()
    src = edge_index[0].reshape(E // ECH, ECH)
    dst = edge_index[1].reshape(E // ECH, ECH)
    dstp = jnp.concatenate(
        [dst, jnp.full((GC, ECH), N, jnp.int32)], axis=0)

    dum = jnp.zeros((ECH, D), jnp.float32)
    deg2 = degk(edge_index[1])

    h1, deg = _first_mm(x, W1, deg2[0, :N], deg2[1, :N])
    p = agg(h1, src, dstp, dum)
    z1, st1 = _comb(p[0, :N], p[1, :N], h1, deg, b1.reshape(1, D))

    h2 = _bn_mm(z1, st1, g1.reshape(1, D), be1.reshape(1, D), W2, deg)
    p = agg(h2, src, dstp, dum)
    z2, st2 = _comb(p[0, :N], p[1, :N], h2, deg, b2.reshape(1, D))

    h3 = _bn_mm(z2, st2, g2.reshape(1, D), be2.reshape(1, D), W3, deg)
    p = agg(h3, src, dstp, dum)
    return _final(p[0, :N], p[1, :N], h3, deg, b3.reshape(1, D))


# speedup vs baseline: 25.9573x; 1.0570x over previous
import functools

import jax
import jax.numpy as jnp
from jax import lax
from jax.experimental import pallas as pl
from jax.experimental.pallas import tpu as pltpu
from jax.experimental.pallas import tpu_sc as plsc

N = 10000
D = 128
E = 320000
EPS = 1e-5

NC = 2
NS = 16
NW = NC * NS
ECH = 125
CPT = (E // NW) // ECH
GC = 8
NGRP = CPT // GC
IR = 4
DCH = 128
NDCH = E // DCH
NPAD = 10240
ROWS_PER_TILE = NPAD // NS
WB = 64
ROWB = ECH * D * 4


BLK = 1000
GRID = N // BLK


def _sc_agg_body(h_hbm, src_hbm, dst_hbm, dum_hbm, out_hbm,
                 acc, sidx_v, didx_v, sbuf0, sbuf1, dbuf0, dbuf1,
                 rows0, rows1, wb_v, isem, gs0, gs1, ss0, ss1):
    gsem = (gs0, gs1)
    ssem = (ss0, ss1)
    rows = (rows0, rows1)
    sbuf = (sbuf0, sbuf1)
    dbuf = (dbuf0, dbuf1)
    cid = lax.axis_index("c")
    sid = lax.axis_index("s")
    wid = cid * NS + sid
    iblk = wid * CPT

    pltpu.async_copy(src_hbm.at[pl.ds(iblk, GC)], sidx_v.at[pl.ds(0, GC)],
                     isem)
    pltpu.async_copy(dst_hbm.at[pl.ds(iblk, GC)], didx_v.at[pl.ds(0, GC)],
                     isem)
    pltpu.async_copy(dst_hbm.at[pl.ds(E // ECH, GC)],
                     didx_v.at[pl.ds(2 * GC, GC)], isem)

    def zrow(j, carry):
        for k in range(D // 16):
            wb_v[j, pl.ds(k * 16, 16)] = jnp.zeros((16,), jnp.float32)
        return carry
    lax.fori_loop(0, WB, zrow, 0)
    base = sid * ROWS_PER_TILE
    for j in range(ROWS_PER_TILE // WB):
        pltpu.sync_copy(wb_v, acc.at[pl.ds(base + j * WB, WB)])

    def iwait(ref):
        pltpu.make_async_copy(src_hbm.at[pl.ds(0, GC)], ref.at[pl.ds(0, GC)],
                              isem).wait()
    iwait(sidx_v)
    iwait(didx_v)
    iwait(didx_v)

    def stage(buf, ref, row):
        for kk in range(GC):
            o = min(kk * 16, ECH - 16)
            buf[pl.ds(o, 16)] = ref[row, pl.ds(o, 16)]

    stage(dbuf0, didx_v, 2 * GC)
    stage(dbuf1, didx_v, 2 * GC)
    stage(sbuf0, sidx_v, 0)
    stage(sbuf1, sidx_v, 0)
    plsc.subcore_barrier()

    def gather(rb):
        pltpu.async_copy(h_hbm.at[sbuf[rb]], rows[rb], gsem[rb])

    def scatter(rb):
        pltpu.async_copy(rows[rb], acc.at[dbuf[rb]], ssem[rb], add=True)

    def gwait(rb):
        pltpu.make_async_copy(dum_hbm, rows[rb], gsem[rb]).wait()

    def swait(rb):
        pltpu.make_async_copy(dum_hbm, rows[rb], ssem[rb]).wait()

    gather(1)
    scatter(0)

    def grp(g, carry):
        sb = (g & 1) * GC
        pb = GC - sb
        gn = jnp.minimum(g + 1, NGRP - 1)
        for k in range(GC):
            b = k & 1
            swait(b)
            if k == 1:
                pltpu.async_copy(src_hbm.at[pl.ds(iblk + GC * gn, GC)],
                                 sidx_v.at[pl.ds(pb, GC)], isem)
                pltpu.async_copy(dst_hbm.at[pl.ds(iblk + GC * gn, GC)],
                                 didx_v.at[pl.ds(pb, GC)], isem)
            stage(sbuf[b], sidx_v, sb + k)
            stage(dbuf[b], didx_v, sb + k)
            gather(b)
            gwait(1 - b)
            scatter(1 - b)
            if k == GC - 1:
                iwait(sidx_v)
                iwait(didx_v)
        return carry
    lax.fori_loop(0, NGRP, grp, 0)

    gwait(1)
    scatter(1)
    swait(0)
    swait(1)
    plsc.subcore_barrier()

    for j in range(ROWS_PER_TILE // WB):
        r = base + j * WB
        pltpu.sync_copy(acc.at[pl.ds(r, WB)], wb_v)
        pltpu.sync_copy(wb_v, out_hbm.at[cid, pl.ds(r, WB)])


def _sc_deg_body(dst_hbm, out_hbm, accd, dst_v, ones_v, wbd_v):
    cid = lax.axis_index("c")
    sid = lax.axis_index("s")
    wid = cid * NS + sid

    def fill_ones(j, carry):
        ones_v[j, pl.ds(0, 16)] = jnp.ones((16,), jnp.float32)
        return carry
    lax.fori_loop(0, DCH, fill_ones, 0)

    def zrow(j, carry):
        wbd_v[j, pl.ds(0, 16)] = jnp.zeros((16,), jnp.float32)
        return carry
    lax.fori_loop(0, WB, zrow, 0)
    base = sid * ROWS_PER_TILE
    for j in range(ROWS_PER_TILE // WB):
        pltpu.sync_copy(wbd_v, accd.at[pl.ds(base + j * WB, WB)])
    plsc.subcore_barrier()

    nchunks = (NDCH - wid + NW - 1) // NW

    def body(i, carry):
        ebase = (wid + i * NW) * DCH
        pltpu.sync_copy(dst_hbm.at[pl.ds(ebase, DCH)], dst_v)
        pltpu.sync_copy(ones_v, accd.at[dst_v], add=True)
        return carry
    lax.fori_loop(0, nchunks, body, 0)
    plsc.subcore_barrier()

    for j in range(ROWS_PER_TILE // WB):
        r = base + j * WB
        pltpu.sync_copy(accd.at[pl.ds(r, WB)], wbd_v)
        pltpu.sync_copy(wbd_v, out_hbm.at[cid, pl.ds(r, WB)])


@functools.lru_cache(maxsize=None)
def _sc_kernels():
    mesh = plsc.VectorSubcoreMesh(core_axis_name="c", subcore_axis_name="s")
    agg = pl.kernel(
        _sc_agg_body,
        out_type=jax.ShapeDtypeStruct((NC, NPAD, D), jnp.float32),
        mesh=mesh,
        scratch_types=[
            pltpu.VMEM_SHARED((NPAD, D), jnp.float32),
            pltpu.VMEM((2 * GC, ECH), jnp.int32),
            pltpu.VMEM((3 * GC, ECH), jnp.int32),
            pltpu.VMEM((ECH,), jnp.int32),
            pltpu.VMEM((ECH,), jnp.int32),
            pltpu.VMEM((ECH,), jnp.int32),
            pltpu.VMEM((ECH,), jnp.int32),
            pltpu.VMEM((ECH, D), jnp.float32),
            pltpu.VMEM((ECH, D), jnp.float32),
            pltpu.VMEM((WB, D), jnp.float32),
        ] + [pltpu.SemaphoreType.DMA] * 5,
    )
    deg = pl.kernel(
        _sc_deg_body,
        out_type=jax.ShapeDtypeStruct((NC, NPAD, 16), jnp.float32),
        mesh=mesh,
        scratch_types=[
            pltpu.VMEM_SHARED((NPAD, 16), jnp.float32),
            pltpu.VMEM((DCH,), jnp.int32),
            pltpu.VMEM((DCH, 16), jnp.float32),
            pltpu.VMEM((WB, 16), jnp.float32),
        ],
    )
    return agg, deg


def _dinv(deg_ref):
    return lax.rsqrt(1.0 + deg_ref[:, :1])


def _first_mm_body(x_ref, w_ref, dd_ref, o_ref, deg_ref):
    deg_ref[...] = dd_ref[0] + dd_ref[1]
    h = jnp.dot(x_ref[...], w_ref[...], preferred_element_type=jnp.float32)
    o_ref[...] = h * _dinv(deg_ref)


def _comb_body(p_ref, h_ref, deg_ref, b_ref, z_ref, st_ref):
    i = pl.program_id(0)
    z = (p_ref[0] + p_ref[1] + h_ref[...]) * _dinv(deg_ref) + b_ref[...]
    z_ref[...] = z

    @pl.when(i == 0)
    def _():
        st_ref[...] = jnp.zeros_like(st_ref)
    s1 = jnp.sum(z, axis=0, keepdims=True)
    s2 = jnp.sum(z * z, axis=0, keepdims=True)
    st_ref[...] += jnp.concatenate(
        [s1, s2, jnp.zeros((6, D), jnp.float32)], axis=0)


def _bn_mm_body(z_ref, st_ref, g_ref, be_ref, w_ref, deg_ref, o_ref):
    mu = st_ref[0:1, :] * (1.0 / N)
    var = st_ref[1:2, :] * (1.0 / N) - mu * mu
    rstd = lax.rsqrt(var + EPS)
    zn = (z_ref[...] - mu) * rstd * g_ref[...] + be_ref[...]
    zn = jnp.maximum(zn, 0.0)
    h = jnp.dot(zn, w_ref[...], preferred_element_type=jnp.float32)
    o_ref[...] = h * _dinv(deg_ref)


def _final_body(p_ref, h_ref, deg_ref, b_ref, o_ref):
    o_ref[...] = ((p_ref[0] + p_ref[1] + h_ref[...]) * _dinv(deg_ref)
                  + b_ref[...])


def _row_spec():
    return pl.BlockSpec((BLK, D), lambda i: (i, 0))


def _pad_spec(w):
    return pl.BlockSpec((2, BLK, w), lambda i: (0, i, 0))


def _deg_spec():
    return pl.BlockSpec((BLK, 16), lambda i: (i, 0))


def _full_spec(rows):
    return pl.BlockSpec((rows, D), lambda i: (0, 0))


def _first_mm(x, w, deg2):
    return pl.pallas_call(
        _first_mm_body,
        grid=(GRID,),
        in_specs=[_row_spec(), _full_spec(D), _pad_spec(16)],
        out_specs=[_row_spec(), _deg_spec()],
        out_shape=[jax.ShapeDtypeStruct((N, D), jnp.float32),
                   jax.ShapeDtypeStruct((N, 16), jnp.float32)],
    )(x, w, deg2)


def _comb(p, h, deg, b):
    return pl.pallas_call(
        _comb_body,
        grid=(GRID,),
        in_specs=[_pad_spec(D), _row_spec(), _deg_spec(), _full_spec(1)],
        out_specs=[_row_spec(), _full_spec(8)],
        out_shape=[jax.ShapeDtypeStruct((N, D), jnp.float32),
                   jax.ShapeDtypeStruct((8, D), jnp.float32)],
    )(p, h, deg, b)


def _bn_mm(z, st, g, be, w, deg):
    return pl.pallas_call(
        _bn_mm_body,
        grid=(GRID,),
        in_specs=[_row_spec(), _full_spec(8), _full_spec(1), _full_spec(1),
                  _full_spec(D), _deg_spec()],
        out_specs=_row_spec(),
        out_shape=jax.ShapeDtypeStruct((N, D), jnp.float32),
    )(z, st, g, be, w, deg)


def _final(p, h, deg, b):
    return pl.pallas_call(
        _final_body,
        grid=(GRID,),
        in_specs=[_pad_spec(D), _row_spec(), _deg_spec(), _full_spec(1)],
        out_specs=_row_spec(),
        out_shape=jax.ShapeDtypeStruct((N, D), jnp.float32),
    )(p, h, deg, b)


def kernel(x, edge_index, W1, b1, g1, be1, W2, b2, g2, be2, W3, b3):
    agg, degk = _sc_kernels()
    src = edge_index[0].reshape(E // ECH, ECH)
    dst = edge_index[1].reshape(E // ECH, ECH)
    dstp = jnp.concatenate(
        [dst, jnp.full((GC, ECH), N, jnp.int32)], axis=0)
    dum = jnp.zeros((ECH, D), jnp.float32)

    deg2 = degk(edge_index[1])

    h1, deg = _first_mm(x, W1, deg2)
    p = agg(h1, src, dstp, dum)
    z1, st1 = _comb(p, h1, deg, b1.reshape(1, D))

    h2 = _bn_mm(z1, st1, g1.reshape(1, D), be1.reshape(1, D), W2, deg)
    p = agg(h2, src, dstp, dum)
    z2, st2 = _comb(p, h2, deg, b2.reshape(1, D))

    h3 = _bn_mm(z2, st2, g2.reshape(1, D), be2.reshape(1, D), W3, deg)
    p = agg(h3, src, dstp, dum)
    return _final(p, h3, deg, b3.reshape(1, D))
